# baseline (speedup 1.0000x reference)
import functools

import jax
import jax.numpy as jnp
from jax.experimental import pallas as pl
from jax.experimental.pallas import tpu as pltpu

NUM_EMB = 1024
EMB_DIM = 64
CCOST = 0.25


def _vq_kernel(x_ref, w_ref, q_ref, enc_ref, loss_ref, perp_ref,
               acc_ref, cnt_ref, wsq_ref, w2_ref,
               *, nsteps, n_tokens, n_elems):
    step = pl.program_id(0) * pl.num_programs(1) + pl.program_id(1)
    w = w_ref[...]

    @pl.when(step == 0)
    def _init():
        acc_ref[0, 0] = 0.0
        cnt_ref[...] = jnp.zeros_like(cnt_ref)
        w2_ref[...] = w + w
        wsq_ref[...] = jnp.sum(w * w, axis=1)[None, :]

    x = x_ref[0]
    wb = x.shape[1]

    s2 = jax.lax.dot_general(x, w2_ref[...], (((0,), (1,)), ((), ())))
    xsq = jnp.sum(x * x, axis=0)
    d = (xsq[:, None] + wsq_ref[...]) - s2
    m = jnp.min(d, axis=1)
    eq = d == m[:, None]
    enc0 = eq.astype(jnp.float32)

    ones = jnp.ones((1, wb), jnp.float32)
    dnc = (((1,), (0,)), ((), ()))
    cnt0 = jax.lax.dot_general(ones, enc0, dnc,
                               preferred_element_type=jnp.float32)
    total = jnp.sum(cnt0)

    @pl.when(total > float(wb))
    def _slow():
        iota = jax.lax.broadcasted_iota(jnp.int32, d.shape, 1)
        idx = jnp.min(jnp.where(eq, iota, NUM_EMB), axis=1)
        enc = (iota == idx[:, None]).astype(jnp.float32)
        enc_ref[0] = enc
        q = jax.lax.dot_general(w, enc, (((0,), (1,)), ((), ())),
                                preferred_element_type=jnp.float32)
        q_ref[0] = q
        acc_ref[0, 0] += jnp.sum((q - x) ** 2)
        cnt_ref[...] += jax.lax.dot_general(ones, enc, dnc,
                                            preferred_element_type=jnp.float32)

    @pl.when(total <= float(wb))
    def _fast():
        enc_ref[0] = enc0
        q = jax.lax.dot_general(w, enc0, (((0,), (1,)), ((), ())),
                                preferred_element_type=jnp.float32)
        q_ref[0] = q
        acc_ref[0, 0] += jnp.sum((q - x) ** 2)
        cnt_ref[...] += cnt0

    @pl.when(step == nsteps - 1)
    def _fini():
        loss_ref[0, 0] = (1.0 + CCOST) * acc_ref[0, 0] / n_elems
        p = cnt_ref[...] / n_tokens
        perp_ref[0, 0] = jnp.exp(-jnp.sum(p * jnp.log(p + 1e-10)))


@jax.jit
def kernel(inputs, embedding_weight):
    b, c, w = inputs.shape
    WB = 1024
    nw = w // WB
    grid = (b, nw)
    n_tokens = b * w
    n_elems = b * w * c

    kfn = functools.partial(_vq_kernel, nsteps=b * nw,
                            n_tokens=float(n_tokens), n_elems=float(n_elems))

    q, enc, loss, perp = pl.pallas_call(
        kfn,
        grid=grid,
        in_specs=[
            pl.BlockSpec((1, c, WB), lambda i, j: (i, 0, j)),
            pl.BlockSpec((NUM_EMB, EMB_DIM), lambda i, j: (0, 0)),
        ],
        out_specs=[
            pl.BlockSpec((1, c, WB), lambda i, j: (i, 0, j)),
            pl.BlockSpec((1, WB, NUM_EMB), lambda i, j: (i, j, 0)),
            pl.BlockSpec((1, 1), lambda i, j: (0, 0),
                         memory_space=pltpu.SMEM),
            pl.BlockSpec((1, 1), lambda i, j: (0, 0),
                         memory_space=pltpu.SMEM),
        ],
        out_shape=[
            jax.ShapeDtypeStruct((b, c, w), jnp.float32),
            jax.ShapeDtypeStruct((b, w, NUM_EMB), jnp.float32),
            jax.ShapeDtypeStruct((1, 1), jnp.float32),
            jax.ShapeDtypeStruct((1, 1), jnp.float32),
        ],
        scratch_shapes=[
            pltpu.SMEM((1, 1), jnp.float32),
            pltpu.VMEM((1, NUM_EMB), jnp.float32),
            pltpu.VMEM((1, NUM_EMB), jnp.float32),
            pltpu.VMEM((NUM_EMB, EMB_DIM), jnp.float32),
        ],
    )(inputs, embedding_weight)

    return q, loss[0, 0], perp[0, 0], enc


# speedup vs baseline: 1.3159x; 1.3159x over previous
import functools

import jax
import jax.numpy as jnp
from jax.experimental import pallas as pl
from jax.experimental.pallas import tpu as pltpu

NUM_EMB = 1024
EMB_DIM = 64
CCOST = 0.25


def _vq_kernel(x_ref, w_ref, q_ref, enc_ref, loss_ref, perp_ref,
               acc_ref, cnt_ref, wsq_ref, w2_ref,
               *, nsteps, n_tokens, n_elems):
    step = pl.program_id(0) * pl.num_programs(1) + pl.program_id(1)
    w = w_ref[...]

    @pl.when(step == 0)
    def _init():
        acc_ref[0, 0] = 0.0
        cnt_ref[...] = jnp.zeros_like(cnt_ref)
        w2_ref[...] = w + w
        wsq_ref[...] = jnp.sum(w * w, axis=1)[None, :]

    x = x_ref[0]
    wb = x.shape[1]

    s2 = jax.lax.dot_general(x, w2_ref[...], (((0,), (1,)), ((), ())))
    xsq = jnp.sum(x * x, axis=0)
    d = (xsq[:, None] + wsq_ref[...]) - s2
    m = jnp.min(d, axis=1)
    iota = jax.lax.broadcasted_iota(jnp.int32, d.shape, 1)
    idx = jnp.min(jnp.where(d == m[:, None], iota, NUM_EMB), axis=1)
    enc = (iota == idx[:, None]).astype(jnp.float32)
    enc_ref[0] = enc

    q = jax.lax.dot_general(w, enc, (((0,), (1,)), ((), ())),
                            preferred_element_type=jnp.float32)
    q_ref[0] = q
    acc_ref[0, 0] += jnp.sum((q - x) ** 2)
    ones = jnp.ones((1, wb), jnp.float32)
    cnt_ref[...] += jax.lax.dot_general(ones, enc, (((1,), (0,)), ((), ())),
                                        preferred_element_type=jnp.float32)

    @pl.when(step == nsteps - 1)
    def _fini():
        loss_ref[0, 0] = (1.0 + CCOST) * acc_ref[0, 0] / n_elems
        p = cnt_ref[...] / n_tokens
        perp_ref[0, 0] = jnp.exp(-jnp.sum(p * jnp.log(p + 1e-10)))


@jax.jit
def kernel(inputs, embedding_weight):
    b, c, w = inputs.shape
    WB = 1024
    nw = w // WB
    grid = (b, nw)
    n_tokens = b * w
    n_elems = b * w * c

    kfn = functools.partial(_vq_kernel, nsteps=b * nw,
                            n_tokens=float(n_tokens), n_elems=float(n_elems))

    q, enc, loss, perp = pl.pallas_call(
        kfn,
        grid=grid,
        in_specs=[
            pl.BlockSpec((1, c, WB), lambda i, j: (i, 0, j)),
            pl.BlockSpec((NUM_EMB, EMB_DIM), lambda i, j: (0, 0)),
        ],
        out_specs=[
            pl.BlockSpec((1, c, WB), lambda i, j: (i, 0, j)),
            pl.BlockSpec((1, WB, NUM_EMB), lambda i, j: (i, j, 0)),
            pl.BlockSpec((1, 1), lambda i, j: (0, 0),
                         memory_space=pltpu.SMEM),
            pl.BlockSpec((1, 1), lambda i, j: (0, 0),
                         memory_space=pltpu.SMEM),
        ],
        out_shape=[
            jax.ShapeDtypeStruct((b, c, w), jnp.float32),
            jax.ShapeDtypeStruct((b, w, NUM_EMB), jnp.float32),
            jax.ShapeDtypeStruct((1, 1), jnp.float32),
            jax.ShapeDtypeStruct((1, 1), jnp.float32),
        ],
        scratch_shapes=[
            pltpu.SMEM((1, 1), jnp.float32),
            pltpu.VMEM((1, NUM_EMB), jnp.float32),
            pltpu.VMEM((1, NUM_EMB), jnp.float32),
            pltpu.VMEM((NUM_EMB, EMB_DIM), jnp.float32),
        ],
    )(inputs, embedding_weight)

    return q, loss[0, 0], perp[0, 0], enc


# speedup vs baseline: 1.4116x; 1.0727x over previous
import functools

import jax
import jax.numpy as jnp
from jax.experimental import pallas as pl
from jax.experimental.pallas import tpu as pltpu

NUM_EMB = 1024
EMB_DIM = 64
CCOST = 0.25


def _vq_kernel(x_ref, w_ref, q_ref, enc_ref, loss_ref, perp_ref,
               acc_ref, cnt_ref, wsq_ref, w2_ref,
               *, nsteps, n_tokens, n_elems):
    step = pl.program_id(0) * pl.num_programs(1) + pl.program_id(1)
    w = w_ref[...]

    @pl.when(step == 0)
    def _init():
        acc_ref[0, 0] = 0.0
        cnt_ref[...] = jnp.zeros_like(cnt_ref)
        w2_ref[...] = w + w
        wsq_ref[...] = jnp.sum(w * w, axis=1)[None, :]

    x = x_ref[0]
    wb = x.shape[1]

    s2 = jax.lax.dot_general(x, w2_ref[...], (((0,), (1,)), ((), ())))
    xsq = jnp.sum(x * x, axis=0)
    d = (xsq[:, None] + wsq_ref[...]) - s2
    m = jnp.min(d, axis=1)
    iota = jax.lax.broadcasted_iota(jnp.int32, d.shape, 1)
    idx = jnp.min(jnp.where(d == m[:, None], iota, NUM_EMB), axis=1)
    enc = (iota == idx[:, None]).astype(jnp.float32)
    enc_ref[0] = enc

    q = jax.lax.dot_general(w, enc, (((0,), (1,)), ((), ())),
                            preferred_element_type=jnp.float32)
    q_ref[0] = q
    acc_ref[0, 0] += jnp.sum((q - x) ** 2)
    ones = jnp.ones((1, wb), jnp.float32)
    cnt_ref[...] += jax.lax.dot_general(ones, enc, (((1,), (0,)), ((), ())),
                                        preferred_element_type=jnp.float32)

    @pl.when(step == nsteps - 1)
    def _fini():
        loss_ref[0, 0] = (1.0 + CCOST) * acc_ref[0, 0] / n_elems
        p = cnt_ref[...] / n_tokens
        perp_ref[0, 0] = jnp.exp(-jnp.sum(p * jnp.log(p + 1e-10)))


@jax.jit
def kernel(inputs, embedding_weight):
    b, c, w = inputs.shape
    WB = 2048
    nw = w // WB
    grid = (b, nw)
    n_tokens = b * w
    n_elems = b * w * c

    kfn = functools.partial(_vq_kernel, nsteps=b * nw,
                            n_tokens=float(n_tokens), n_elems=float(n_elems))

    q, enc, loss, perp = pl.pallas_call(
        kfn,
        grid=grid,
        in_specs=[
            pl.BlockSpec((1, c, WB), lambda i, j: (i, 0, j)),
            pl.BlockSpec((NUM_EMB, EMB_DIM), lambda i, j: (0, 0)),
        ],
        out_specs=[
            pl.BlockSpec((1, c, WB), lambda i, j: (i, 0, j)),
            pl.BlockSpec((1, WB, NUM_EMB), lambda i, j: (i, j, 0)),
            pl.BlockSpec((1, 1), lambda i, j: (0, 0),
                         memory_space=pltpu.SMEM),
            pl.BlockSpec((1, 1), lambda i, j: (0, 0),
                         memory_space=pltpu.SMEM),
        ],
        out_shape=[
            jax.ShapeDtypeStruct((b, c, w), jnp.float32),
            jax.ShapeDtypeStruct((b, w, NUM_EMB), jnp.float32),
            jax.ShapeDtypeStruct((1, 1), jnp.float32),
            jax.ShapeDtypeStruct((1, 1), jnp.float32),
        ],
        scratch_shapes=[
            pltpu.SMEM((1, 1), jnp.float32),
            pltpu.VMEM((1, NUM_EMB), jnp.float32),
            pltpu.VMEM((1, NUM_EMB), jnp.float32),
            pltpu.VMEM((NUM_EMB, EMB_DIM), jnp.float32),
        ],
    )(inputs, embedding_weight)

    return q, loss[0, 0], perp[0, 0], enc
